# baseline (speedup 1.0000x reference)
import jax
import jax.numpy as jnp
from jax.experimental import pallas as pl
from jax.experimental.pallas import tpu as pltpu

B = 16
N = 16384
D = 256
H = 512
TILE = 2048


def _qnet_kernel(ps_ref, gemb_ref, w1_ref, b1_ref, w2_ref, b2_ref,
                 embed_ref, out_ref, g_scratch, w1a_scratch):
    i = pl.program_id(0)

    @pl.when(i == 0)
    def _():
        w1a_scratch[...] = w1_ref[:D, :].astype(jnp.bfloat16)
        g = jnp.dot(gemb_ref[...].astype(jnp.bfloat16),
                    w1_ref[D:, :].astype(jnp.bfloat16),
                    preferred_element_type=jnp.float32)
        g_scratch[...] = (g + b1_ref[...]).astype(jnp.bfloat16)

    rows = i * TILE + jax.lax.broadcasted_iota(jnp.int32, (TILE, B), 0)
    starts = ps_ref[0:1, :]
    ends = ps_ref[1:2, :]
    onehot = (rows >= starts) & (rows < ends)

    acc = jnp.dot(embed_ref[...].astype(jnp.bfloat16), w1a_scratch[...],
                  preferred_element_type=jnp.float32)
    acc = acc + jnp.dot(onehot.astype(jnp.bfloat16), g_scratch[...],
                        preferred_element_type=jnp.float32)
    h = jnp.maximum(acc, 0.0).astype(jnp.bfloat16)
    out_ref[...] = jnp.dot(h, w2_ref[...].astype(jnp.bfloat16),
                           preferred_element_type=jnp.float32) + b2_ref[0, 0]


@jax.jit
def kernel(embed, graph_embed, prefix_sum, W1, b1, W2, b2):
    grid = (N // TILE,)
    out_fn = pl.pallas_call(
        _qnet_kernel,
        grid=grid,
        in_specs=[
            pl.BlockSpec((2, B), lambda i: (0, 0)),
            pl.BlockSpec((B, D), lambda i: (0, 0)),
            pl.BlockSpec((2 * D, H), lambda i: (0, 0)),
            pl.BlockSpec((1, H), lambda i: (0, 0)),
            pl.BlockSpec((H, 1), lambda i: (0, 0)),
            pl.BlockSpec((1, 1), lambda i: (0, 0)),
            pl.BlockSpec((TILE, D), lambda i: (i, 0)),
        ],
        out_specs=pl.BlockSpec((TILE, 1), lambda i: (i, 0)),
        out_shape=jax.ShapeDtypeStruct((N, 1), jnp.float32),
        scratch_shapes=[pltpu.VMEM((B, H), jnp.bfloat16),
                        pltpu.VMEM((D, H), jnp.bfloat16)],
    )
    ends2d = prefix_sum.reshape(1, B)
    starts2d = jnp.concatenate(
        [jnp.zeros((1, 1), jnp.int32), ends2d[:, :-1]], axis=1)
    ps2d = jnp.concatenate([starts2d, ends2d], axis=0)
    return out_fn(ps2d, graph_embed, W1, b1.reshape(1, H),
                  W2, b2.reshape(1, 1), embed)


# speedup vs baseline: 1.0363x; 1.0363x over previous
import jax
import jax.numpy as jnp
from jax.experimental import pallas as pl
from jax.experimental.pallas import tpu as pltpu

B = 16
N = 16384
D = 256
H = 512
TILE = 2048


def _qnet_kernel(ps_ref, gemb_ref, w1_ref, b1_ref, w2_ref, b2_ref,
                 embed_ref, out_ref, g_scratch, w1a_scratch):
    i = pl.program_id(0)

    @pl.when(i == 0)
    def _():
        w1a_scratch[...] = w1_ref[:D, :].astype(jnp.bfloat16)
        g = jnp.dot(gemb_ref[...].astype(jnp.bfloat16),
                    w1_ref[D:, :].astype(jnp.bfloat16),
                    preferred_element_type=jnp.float32)
        g_scratch[...] = (g + b1_ref[...]).astype(jnp.bfloat16)

    rows = i * TILE + jax.lax.broadcasted_iota(jnp.int32, (TILE, B), 0)
    ends = ps_ref[...]
    starts = jnp.concatenate(
        [jnp.zeros((1, 1), jnp.int32), ends[:, :B - 1]], axis=1)
    onehot = (rows >= starts) & (rows < ends)

    acc = jnp.dot(embed_ref[...].astype(jnp.bfloat16), w1a_scratch[...],
                  preferred_element_type=jnp.float32)
    acc = acc + jnp.dot(onehot.astype(jnp.bfloat16), g_scratch[...],
                        preferred_element_type=jnp.float32)
    h = jnp.maximum(acc, 0.0).astype(jnp.bfloat16)
    out_ref[...] = jnp.dot(h, w2_ref[...].astype(jnp.bfloat16),
                           preferred_element_type=jnp.float32) + b2_ref[0, 0]


@jax.jit
def kernel(embed, graph_embed, prefix_sum, W1, b1, W2, b2):
    grid = (N // TILE,)
    out_fn = pl.pallas_call(
        _qnet_kernel,
        grid=grid,
        in_specs=[
            pl.BlockSpec((1, B), lambda i: (0, 0)),
            pl.BlockSpec((B, D), lambda i: (0, 0)),
            pl.BlockSpec((2 * D, H), lambda i: (0, 0)),
            pl.BlockSpec((1, H), lambda i: (0, 0)),
            pl.BlockSpec((H, 1), lambda i: (0, 0)),
            pl.BlockSpec((1, 1), lambda i: (0, 0)),
            pl.BlockSpec((TILE, D), lambda i: (i, 0)),
        ],
        out_specs=pl.BlockSpec((TILE, 1), lambda i: (i, 0)),
        out_shape=jax.ShapeDtypeStruct((N, 1), jnp.float32),
        scratch_shapes=[pltpu.VMEM((B, H), jnp.bfloat16),
                        pltpu.VMEM((D, H), jnp.bfloat16)],
    )
    return out_fn(prefix_sum.reshape(1, B), graph_embed, W1,
                  b1.reshape(1, H), W2, b2.reshape(1, 1), embed)


# speedup vs baseline: 1.0620x; 1.0248x over previous
import jax
import jax.numpy as jnp
from jax.experimental import pallas as pl
from jax.experimental.pallas import tpu as pltpu

B = 16
N = 16384
D = 256
H = 512
TILE = 4096


def _qnet_kernel(ps_ref, gemb_ref, w1_ref, b1_ref, w2_ref, b2_ref,
                 embed_ref, out_ref, g_scratch, w1a_scratch):
    i = pl.program_id(0)

    @pl.when(i == 0)
    def _():
        w1a_scratch[...] = w1_ref[:D, :].astype(jnp.bfloat16)
        g = jnp.dot(gemb_ref[...].astype(jnp.bfloat16),
                    w1_ref[D:, :].astype(jnp.bfloat16),
                    preferred_element_type=jnp.float32)
        g_scratch[...] = (g + b1_ref[...]).astype(jnp.bfloat16)

    rows = i * TILE + jax.lax.broadcasted_iota(jnp.int32, (TILE, B), 0)
    ends = ps_ref[...]
    starts = jnp.concatenate(
        [jnp.zeros((1, 1), jnp.int32), ends[:, :B - 1]], axis=1)
    onehot = (rows >= starts) & (rows < ends)

    acc = jnp.dot(embed_ref[...].astype(jnp.bfloat16), w1a_scratch[...],
                  preferred_element_type=jnp.float32)
    acc = acc + jnp.dot(onehot.astype(jnp.bfloat16), g_scratch[...],
                        preferred_element_type=jnp.float32)
    h = jnp.maximum(acc, 0.0).astype(jnp.bfloat16)
    out_ref[...] = jnp.dot(h, w2_ref[...].astype(jnp.bfloat16),
                           preferred_element_type=jnp.float32) + b2_ref[0, 0]


@jax.jit
def kernel(embed, graph_embed, prefix_sum, W1, b1, W2, b2):
    grid = (N // TILE,)
    out_fn = pl.pallas_call(
        _qnet_kernel,
        grid=grid,
        in_specs=[
            pl.BlockSpec((1, B), lambda i: (0, 0)),
            pl.BlockSpec((B, D), lambda i: (0, 0)),
            pl.BlockSpec((2 * D, H), lambda i: (0, 0)),
            pl.BlockSpec((1, H), lambda i: (0, 0)),
            pl.BlockSpec((H, 1), lambda i: (0, 0)),
            pl.BlockSpec((1, 1), lambda i: (0, 0)),
            pl.BlockSpec((TILE, D), lambda i: (i, 0)),
        ],
        out_specs=pl.BlockSpec((TILE, 1), lambda i: (i, 0)),
        out_shape=jax.ShapeDtypeStruct((N, 1), jnp.float32),
        scratch_shapes=[pltpu.VMEM((B, H), jnp.bfloat16),
                        pltpu.VMEM((D, H), jnp.bfloat16)],
    )
    return out_fn(prefix_sum.reshape(1, B), graph_embed, W1,
                  b1.reshape(1, H), W2, b2.reshape(1, 1), embed)


# speedup vs baseline: 1.0645x; 1.0023x over previous
import jax
import jax.numpy as jnp
from jax.experimental import pallas as pl
from jax.experimental.pallas import tpu as pltpu

B = 16
N = 16384
D = 256
H = 512
TILE = 4096


def _qnet_kernel(ps_ref, gemb_ref, w1_ref, b1_ref, w2_ref, b2_ref,
                 embed_ref, out_ref, waug_scratch):
    i = pl.program_id(0)

    @pl.when(i == 0)
    def _():
        waug_scratch[:D, :] = w1_ref[:D, :].astype(jnp.bfloat16)
        g = jnp.dot(gemb_ref[...].astype(jnp.bfloat16),
                    w1_ref[D:, :].astype(jnp.bfloat16),
                    preferred_element_type=jnp.float32)
        waug_scratch[D:, :] = (g + b1_ref[...]).astype(jnp.bfloat16)

    rows = i * TILE + jax.lax.broadcasted_iota(jnp.int32, (TILE, B), 0)
    ends = ps_ref[...]
    starts = jnp.concatenate(
        [jnp.zeros((1, 1), jnp.int32), ends[:, :B - 1]], axis=1)
    onehot = ((rows >= starts) & (rows < ends)).astype(jnp.bfloat16)

    x_aug = jnp.concatenate(
        [embed_ref[...].astype(jnp.bfloat16), onehot], axis=1)
    acc = jnp.dot(x_aug, waug_scratch[...],
                  preferred_element_type=jnp.float32)
    h = jnp.maximum(acc.astype(jnp.bfloat16), jnp.bfloat16(0.0))
    out_ref[...] = jnp.dot(h, w2_ref[...].astype(jnp.bfloat16),
                           preferred_element_type=jnp.float32) + b2_ref[0, 0]


@jax.jit
def kernel(embed, graph_embed, prefix_sum, W1, b1, W2, b2):
    grid = (N // TILE,)
    out_fn = pl.pallas_call(
        _qnet_kernel,
        grid=grid,
        in_specs=[
            pl.BlockSpec((1, B), lambda i: (0, 0)),
            pl.BlockSpec((B, D), lambda i: (0, 0)),
            pl.BlockSpec((2 * D, H), lambda i: (0, 0)),
            pl.BlockSpec((1, H), lambda i: (0, 0)),
            pl.BlockSpec((H, 1), lambda i: (0, 0)),
            pl.BlockSpec((1, 1), lambda i: (0, 0)),
            pl.BlockSpec((TILE, D), lambda i: (i, 0)),
        ],
        out_specs=pl.BlockSpec((TILE, 1), lambda i: (i, 0)),
        out_shape=jax.ShapeDtypeStruct((N, 1), jnp.float32),
        scratch_shapes=[pltpu.VMEM((D + B, H), jnp.bfloat16)],
    )
    return out_fn(prefix_sum.reshape(1, B), graph_embed, W1,
                  b1.reshape(1, H), W2, b2.reshape(1, 1), embed)
